# baseline (speedup 1.0000x reference)
import functools

import jax
import jax.numpy as jnp
from jax import lax
from jax.experimental import pallas as pl
from jax.experimental.pallas import tpu as pltpu
from jax.experimental.pallas import tpu_sc as plsc

DP = 256
BLK = 128


@functools.lru_cache(maxsize=None)
def _make_gather(B, S, V, D):
    info = plsc.get_sparse_core_info()
    NC, NS = info.num_cores, info.num_subcores
    NW = NC * NS
    assert B == NW * BLK and D % 8 == 0 and S % 8 == 0
    NT = D // 8
    NG = S // 8
    mesh = plsc.VectorSubcoreMesh(core_axis_name="c", subcore_axis_name="s")

    @functools.partial(
        pl.kernel,
        mesh=mesh,
        out_type=jax.ShapeDtypeStruct((S, NT, 8, B), jnp.float32),
        scratch_types=[
            pltpu.VMEM((8, BLK), jnp.int32),
            pltpu.VMEM((BLK, DP), jnp.float32),
            pltpu.VMEM((BLK, DP), jnp.float32),
            pltpu.VMEM((NT, 8, BLK), jnp.float32),
            pltpu.VMEM((NT, 8, BLK), jnp.float32),
            pltpu.SemaphoreType.DMA,
            pltpu.SemaphoreType.DMA,
            pltpu.SemaphoreType.DMA,
            pltpu.SemaphoreType.DMA,
        ],
        compiler_params=pltpu.CompilerParams(needs_layout_passes=False),
    )
    def gather_kernel(xt_hbm, table_hbm, out_hbm, sidx, rows0, rows1,
                      ob0, ob1, gs0, gs1, ws0, ws1):
        wid = lax.axis_index("s") * NC + lax.axis_index("c")
        col = wid * BLK
        rows = (rows0, rows1)
        obs = (ob0, ob1)
        gsems = (gs0, gs1)
        wsems = (ws0, ws1)
        lane = lax.iota(jnp.int32, 16)
        rvecs = [lane + m * 16 for m in range(8)]

        def load_sidx(sg):
            pltpu.sync_copy(
                xt_hbm.at[pl.ds(sg * 8, 8), pl.ds(col, BLK)], sidx)

        def start_gather(r, b):
            pltpu.async_copy(table_hbm.at[sidx.at[r]], rows[b], gsems[b])

        def wait_gather(r, b):
            pltpu.make_async_copy(
                table_hbm.at[sidx.at[r]], rows[b], gsems[b]).wait()

        def start_write(s, b):
            pltpu.async_copy(
                obs[b], out_hbm.at[s, :, :, pl.ds(col, BLK)], wsems[b])

        def wait_write(s, b):
            pltpu.make_async_copy(
                obs[b], out_hbm.at[s, :, :, pl.ds(col, BLK)], wsems[b]).wait()

        def transpose(b):
            rb, ob = rows[b], obs[b]

            def tbody(dt, carry):
                for ds in range(8):
                    cvec = lane * 0 + (dt * 8 + ds)
                    for m in range(8):
                        v = plsc.load_gather(rb, [rvecs[m], cvec])
                        ob[dt, ds, pl.ds(m * 16, 16)] = v
                return carry

            lax.fori_loop(0, NT, tbody, 0)

        load_sidx(0)
        start_gather(0, 0)
        wait_gather(0, 0)
        start_gather(1, 1)
        transpose(0)
        start_write(0, 0)
        wait_gather(1, 1)
        start_gather(2, 0)
        transpose(1)
        start_write(1, 1)

        def pbody(p, carry):
            s0 = 2 * p
            s1 = s0 + 1
            wait_gather(s0 % 8, 0)
            start_gather(s1 % 8, 1)
            wait_write(s0 - 2, 0)
            transpose(0)
            start_write(s0, 0)
            wait_gather(s1 % 8, 1)

            @pl.when(jnp.logical_and(p < S // 2 - 1, (s1 + 1) % 8 == 0))
            def _():
                load_sidx((s1 + 1) // 8)

            @pl.when(p < S // 2 - 1)
            def _():
                start_gather((s1 + 1) % 8, 0)

            wait_write(s1 - 2, 1)
            transpose(1)
            start_write(s1, 1)
            return carry

        lax.fori_loop(1, S // 2, pbody, 0)
        wait_write(S - 2, 0)
        wait_write(S - 1, 1)

    return gather_kernel


def kernel(x, table):
    B, S = x.shape
    V, D = table.shape
    xt = x.T
    table_p = jnp.pad(table, ((0, 0), (0, DP - D)))
    out2 = _make_gather(B, S, V, D)(xt, table_p)
    return out2.transpose(3, 0, 1, 2).reshape(B, S, D)


# speedup vs baseline: 1.2229x; 1.2229x over previous
import functools

import jax
import jax.numpy as jnp
from jax import lax
from jax.experimental import pallas as pl
from jax.experimental.pallas import tpu as pltpu
from jax.experimental.pallas import tpu_sc as plsc

DP = 256
BLK = 128


@functools.lru_cache(maxsize=None)
def _make_gather(B, S, V, D):
    info = plsc.get_sparse_core_info()
    NC, NS = info.num_cores, info.num_subcores
    NW = NC * NS
    assert B == NW * BLK and D % 8 == 0 and S % 8 == 0
    NT = D // 8
    NG = S // 8
    mesh = plsc.VectorSubcoreMesh(core_axis_name="c", subcore_axis_name="s")

    @functools.partial(
        pl.kernel,
        mesh=mesh,
        out_type=jax.ShapeDtypeStruct((S, NT, 8, B), jnp.float32),
        scratch_types=[
            pltpu.VMEM((8, BLK), jnp.int32),
            pltpu.VMEM((BLK, DP), jnp.float32),
            pltpu.VMEM((BLK, DP), jnp.float32),
            pltpu.VMEM((NT, 8, BLK), jnp.float32),
            pltpu.VMEM((NT, 8, BLK), jnp.float32),
            pltpu.SemaphoreType.DMA,
            pltpu.SemaphoreType.DMA,
            pltpu.SemaphoreType.DMA,
            pltpu.SemaphoreType.DMA,
        ],
        compiler_params=pltpu.CompilerParams(needs_layout_passes=False),
    )
    def gather_kernel(xt_hbm, table_hbm, out_hbm, sidx, rows0, rows1,
                      ob0, ob1, gs0, gs1, ws0, ws1):
        wid = lax.axis_index("s") * NC + lax.axis_index("c")
        col = wid * BLK
        rows = (rows0, rows1)
        obs = (ob0, ob1)
        gsems = (gs0, gs1)
        wsems = (ws0, ws1)
        lane = lax.iota(jnp.int32, 16)
        rvecs = [lane + m * 16 for m in range(8)]

        def load_sidx(sg):
            pltpu.sync_copy(
                xt_hbm.at[pl.ds(sg * 8, 8), pl.ds(col, BLK)], sidx)

        def start_gather(r, b):
            pltpu.async_copy(table_hbm.at[sidx.at[r]], rows[b], gsems[b])

        def wait_gather(r, b):
            pltpu.make_async_copy(
                table_hbm.at[sidx.at[r]], rows[b], gsems[b]).wait()

        def start_write(s, b):
            pltpu.async_copy(
                obs[b], out_hbm.at[s, :, :, pl.ds(col, BLK)], wsems[b])

        def wait_write(s, b):
            pltpu.make_async_copy(
                obs[b], out_hbm.at[s, :, :, pl.ds(col, BLK)], wsems[b]).wait()

        def transpose(b):
            rb, ob = rows[b], obs[b]

            def tbody(dt, carry):
                for ds in range(0, 8, 2):
                    cv0 = lane * 0 + (dt * 8 + ds)
                    cv1 = lane * 0 + (dt * 8 + ds + 1)
                    vs = [plsc.load_gather(rb, [rvecs[m], cv0])
                          for m in range(8)]
                    vs += [plsc.load_gather(rb, [rvecs[m], cv1])
                           for m in range(8)]
                    for m in range(8):
                        ob[dt, ds, pl.ds(m * 16, 16)] = vs[m]
                    for m in range(8):
                        ob[dt, ds + 1, pl.ds(m * 16, 16)] = vs[8 + m]
                return carry

            lax.fori_loop(0, NT, tbody, 0)

        load_sidx(0)
        start_gather(0, 0)
        wait_gather(0, 0)
        start_gather(1, 1)
        transpose(0)
        start_write(0, 0)
        wait_gather(1, 1)
        start_gather(2, 0)
        transpose(1)
        start_write(1, 1)

        def pbody(p, carry):
            s0 = 2 * p
            s1 = s0 + 1
            wait_gather(s0 % 8, 0)
            start_gather(s1 % 8, 1)
            wait_write(s0 - 2, 0)
            transpose(0)
            start_write(s0, 0)
            wait_gather(s1 % 8, 1)

            @pl.when(jnp.logical_and(p < S // 2 - 1, (s1 + 1) % 8 == 0))
            def _():
                load_sidx((s1 + 1) // 8)

            @pl.when(p < S // 2 - 1)
            def _():
                start_gather((s1 + 1) % 8, 0)

            wait_write(s1 - 2, 1)
            transpose(1)
            start_write(s1, 1)
            return carry

        lax.fori_loop(1, S // 2, pbody, 0)
        wait_write(S - 2, 0)
        wait_write(S - 1, 1)

    return gather_kernel


def kernel(x, table):
    B, S = x.shape
    V, D = table.shape
    xt = x.T
    table_p = jnp.pad(table, ((0, 0), (0, DP - D)))
    out2 = _make_gather(B, S, V, D)(xt, table_p)
    return out2.transpose(3, 0, 1, 2).reshape(B, S, D)


# speedup vs baseline: 2.6992x; 2.2073x over previous
import functools

import jax
import jax.numpy as jnp
from jax import lax
from jax.experimental import pallas as pl
from jax.experimental.pallas import tpu as pltpu
from jax.experimental.pallas import tpu_sc as plsc

CHUNK = 128
DP = 256


@functools.lru_cache(maxsize=None)
def _make_gather(B, V):
    info = plsc.get_sparse_core_info()
    NC, NS = info.num_cores, info.num_subcores
    NW = NC * NS
    assert B % (NW * CHUNK) == 0
    b_per_w = B // NW
    n_chunks = b_per_w // CHUNK
    assert n_chunks % 3 == 2
    mesh = plsc.VectorSubcoreMesh(core_axis_name="c", subcore_axis_name="s")

    @functools.partial(
        pl.kernel,
        mesh=mesh,
        out_type=jax.ShapeDtypeStruct((B, DP), jnp.float32),
        scratch_types=[
            pltpu.VMEM((b_per_w,), jnp.int32),
            pltpu.VMEM((CHUNK, DP), jnp.float32),
            pltpu.VMEM((CHUNK, DP), jnp.float32),
            pltpu.VMEM((CHUNK, DP), jnp.float32),
            pltpu.SemaphoreType.DMA,
            pltpu.SemaphoreType.DMA,
            pltpu.SemaphoreType.DMA,
            pltpu.SemaphoreType.DMA,
            pltpu.SemaphoreType.DMA,
            pltpu.SemaphoreType.DMA,
        ],
    )
    def gather_kernel(idx_hbm, table_hbm, out_hbm, idx_v, rows0, rows1,
                      rows2, gs0, gs1, gs2, ss0, ss1, ss2):
        wid = lax.axis_index("s") * NC + lax.axis_index("c")
        base = wid * b_per_w
        pltpu.sync_copy(idx_hbm.at[pl.ds(base, b_per_w)], idx_v)
        rows = (rows0, rows1, rows2)
        gsems = (gs0, gs1, gs2)
        ssems = (ss0, ss1, ss2)

        def start_gather(c, b):
            pltpu.async_copy(
                table_hbm.at[idx_v.at[pl.ds(c * CHUNK, CHUNK)]], rows[b],
                gsems[b])

        def wait_gather(c, b):
            pltpu.make_async_copy(
                table_hbm.at[idx_v.at[pl.ds(c * CHUNK, CHUNK)]], rows[b],
                gsems[b]).wait()

        def start_scatter(c, b):
            pltpu.async_copy(
                rows[b], out_hbm.at[pl.ds(base + c * CHUNK, CHUNK), :],
                ssems[b])

        def wait_scatter(c, b):
            pltpu.make_async_copy(
                rows[b], out_hbm.at[pl.ds(base + c * CHUNK, CHUNK), :],
                ssems[b]).wait()

        for r in range(3):
            start_gather(r, r)
        for r in range(3):
            wait_gather(r, r)
            start_scatter(r, r)

        def body(t, carry):
            c = 3 * t
            for r in range(3):
                wait_scatter(c + r - 3, r)
                start_gather(c + r, r)
            for r in range(3):
                wait_gather(c + r, r)
                start_scatter(c + r, r)
            return carry

        lax.fori_loop(1, (n_chunks - 2) // 3, body, 0)
        for r in range(2):
            c = n_chunks - 2 + r
            wait_scatter(c - 3, r)
            start_gather(c, r)
        for r in range(2):
            c = n_chunks - 2 + r
            wait_gather(c, r)
            start_scatter(c, r)
        wait_scatter(n_chunks - 3, 2)
        wait_scatter(n_chunks - 2, 0)
        wait_scatter(n_chunks - 1, 1)

    return gather_kernel


def kernel(x, table):
    B, S = x.shape
    V, D = table.shape
    flat = x.reshape(B * S).astype(jnp.int32)
    table_p = jnp.pad(table, ((0, 0), (0, DP - D)))
    out = _make_gather(B * S, V)(flat, table_p)
    return out[:, :D].reshape(B, S, D)
